# baseline (speedup 1.0000x reference)
import functools

import jax
import jax.numpy as jnp
from jax import lax
from jax.experimental import pallas as pl
from jax.experimental.pallas import tpu as pltpu
from jax.experimental.pallas import tpu_sc as plsc

RADIUS = 100.0
K = 64
NS = 2048
TB = 256


def _topk_kernel(t8_ref, sT8_ref, idx_ref, dist_ref, d_scratch):
    t8 = t8_ref[0]
    sT8 = sT8_ref[0]

    mm = jnp.dot(t8.astype(jnp.bfloat16), sT8.astype(jnp.bfloat16),
                 preferred_element_type=jnp.float32)
    tx, ty, tz = t8[:, 0:1], t8[:, 1:2], t8[:, 2:3]
    r0 = (tx * tx + ty * ty) + tz * tz
    sx, sy, sz = sT8[0:1, :], sT8[1:2, :], sT8[2:3, :]
    r1 = (sx * sx + sy * sy) + sz * sz
    d_scratch[...] = (r0 - 2.0 * mm) + r1

    iota = jax.lax.broadcasted_iota(jnp.int32, (TB, NS), 1)
    liota = jax.lax.broadcasted_iota(jnp.int32, (TB, K), 1)

    def body(i, prev_idx):
        d = jnp.where(iota == prev_idx, jnp.inf, d_scratch[...])
        d_scratch[...] = d
        vmin = jnp.min(d, axis=1, keepdims=True)
        idx = jnp.min(jnp.where(d == vmin, iota, NS), axis=1,
                      keepdims=True)

        masked = vmin > RADIUS * RADIUS
        here = liota == i
        idx_ref[0] = jnp.where(here, jnp.where(masked, -1, idx), idx_ref[0])
        dist_ref[0] = jnp.where(
            here, jnp.sqrt(jnp.maximum(vmin, 1e-9)) / RADIUS, dist_ref[0])
        return idx

    jax.lax.fori_loop(0, K, body, jnp.full((TB, 1), NS, jnp.int32))


def _make_sc_gather(M, NT, n_workers, num_cores):
    per_w = M // n_workers
    CH = 2048
    n_chunks = per_w // CH
    steps = CH // 16
    f32 = jnp.float32
    mesh = plsc.VectorSubcoreMesh(core_axis_name="c", subcore_axis_name="s")

    @functools.partial(
        pl.kernel, mesh=mesh,
        compiler_params=pltpu.CompilerParams(needs_layout_passes=False),
        out_type=[jax.ShapeDtypeStruct((M,), f32)] * 3,
        scratch_types=(
            [pltpu.VMEM((NT,), f32) for _ in range(6)]
            + [pltpu.VMEM((CH,), jnp.int32)]
            + [pltpu.VMEM((CH,), f32) for _ in range(3)]
        ),
    )
    def sc_gather(gidx_hbm, xt_hbm, yt_hbm, zt_hbm, tnx_hbm, tny_hbm,
                  tnz_hbm, px_hbm, py_hbm, pz_hbm,
                  xt_v, yt_v, zt_v, tnx_v, tny_v, tnz_v,
                  idx_v, ox_v, oy_v, oz_v):
        wid = lax.axis_index("s") * num_cores + lax.axis_index("c")
        base = wid * per_w
        pltpu.sync_copy(xt_hbm, xt_v)
        pltpu.sync_copy(yt_hbm, yt_v)
        pltpu.sync_copy(zt_hbm, zt_v)
        pltpu.sync_copy(tnx_hbm, tnx_v)
        pltpu.sync_copy(tny_hbm, tny_v)
        pltpu.sync_copy(tnz_hbm, tnz_v)
        vio = lax.iota(jnp.int32, 16)

        def chunk_body(ci, _):
            cbase = base + ci * CH
            pltpu.sync_copy(gidx_hbm.at[pl.ds(cbase, CH)], idx_v)

            def step(j, _):
                off = j * 16
                iv = idx_v[pl.ds(off, 16)]
                tidx = ((cbase + off) + vio) >> 6
                ox_v[pl.ds(off, 16)] = (
                    plsc.load_gather(xt_v, [iv])
                    - plsc.load_gather(tnx_v, [tidx]))
                oy_v[pl.ds(off, 16)] = (
                    plsc.load_gather(yt_v, [iv])
                    - plsc.load_gather(tny_v, [tidx]))
                oz_v[pl.ds(off, 16)] = (
                    plsc.load_gather(zt_v, [iv])
                    - plsc.load_gather(tnz_v, [tidx]))
                return 0

            lax.fori_loop(0, steps, step, 0)
            pltpu.sync_copy(ox_v, px_hbm.at[pl.ds(cbase, CH)])
            pltpu.sync_copy(oy_v, py_hbm.at[pl.ds(cbase, CH)])
            pltpu.sync_copy(oz_v, pz_hbm.at[pl.ds(cbase, CH)])
            return 0

        lax.fori_loop(0, n_chunks, chunk_body, 0)

    return sc_gather


@jax.jit
def kernel(source_points, target_points):
    B, NT, _ = target_points.shape
    f32 = jnp.float32

    pad5 = jnp.zeros((B, NT, 5), f32)
    t8 = jnp.concatenate([target_points, pad5], axis=2)
    sT = jnp.transpose(source_points, (0, 2, 1))
    sT8 = jnp.concatenate([sT, jnp.zeros((B, 5, NS), f32)], axis=1)

    grid = (B, NT // TB)
    out_spec = pl.BlockSpec((1, TB, K), lambda b, tb: (b, tb, 0))
    idx, dist = pl.pallas_call(
        _topk_kernel,
        grid=grid,
        in_specs=[
            pl.BlockSpec((1, TB, 8), lambda b, tb: (b, tb, 0)),
            pl.BlockSpec((1, 8, NS), lambda b, tb: (b, 0, 0)),
        ],
        out_specs=[out_spec] * 2,
        out_shape=[
            jax.ShapeDtypeStruct((B, NT, K), jnp.int32),
            jax.ShapeDtypeStruct((B, NT, K), f32),
        ],
        scratch_shapes=[pltpu.VMEM((TB, NS), f32)],
    )(t8, sT8)

    bb = jnp.arange(B, dtype=jnp.int32).reshape(B, 1, 1)
    gidx = (bb * NS + (idx & (NS - 1))).reshape(-1)

    source_n = source_points / RADIUS
    target_n = target_points / RADIUS
    xt = source_n[..., 0].reshape(-1)
    yt = source_n[..., 1].reshape(-1)
    zt = source_n[..., 2].reshape(-1)
    tnx = target_n[..., 0].reshape(-1)
    tny = target_n[..., 1].reshape(-1)
    tnz = target_n[..., 2].reshape(-1)

    M = B * NT * K
    info = plsc.get_sparse_core_info()
    n_workers = info.num_cores * info.num_subcores
    sc_gather = _make_sc_gather(M, B * NS, n_workers, info.num_cores)
    px, py, pz = sc_gather(gidx, xt, yt, zt, tnx, tny, tnz)

    patches = jnp.stack(
        [px.reshape(B, NT, K), py.reshape(B, NT, K), pz.reshape(B, NT, K)],
        axis=-1)
    rad = jnp.full((B, 1, 1), RADIUS, f32)
    patches_size = jnp.full((B, NT), float(K), f32)
    return patches, idx, patches_size, rad, dist


# speedup vs baseline: 1.0675x; 1.0675x over previous
import functools

import jax
import jax.numpy as jnp
from jax import lax
from jax.experimental import pallas as pl
from jax.experimental.pallas import tpu as pltpu
from jax.experimental.pallas import tpu_sc as plsc

RADIUS = 100.0
K = 64
NS = 2048
TB = 256


def _topk_kernel(t8_ref, sT8_ref, idx_ref, dist_ref, d_scratch):
    t8 = t8_ref[0]
    sT8 = sT8_ref[0]

    mm = jnp.dot(t8.astype(jnp.bfloat16), sT8.astype(jnp.bfloat16),
                 preferred_element_type=jnp.float32)
    tx, ty, tz = t8[:, 0:1], t8[:, 1:2], t8[:, 2:3]
    r0 = (tx * tx + ty * ty) + tz * tz
    sx, sy, sz = sT8[0:1, :], sT8[1:2, :], sT8[2:3, :]
    r1 = (sx * sx + sy * sy) + sz * sz
    d_scratch[...] = (r0 - 2.0 * mm) + r1

    iota = jax.lax.broadcasted_iota(jnp.int32, (TB, NS), 1)
    liota = jax.lax.broadcasted_iota(jnp.int32, (TB, K), 1)

    def extract(j, d):
        vmin = jnp.min(d, axis=1, keepdims=True)
        idx = jnp.min(jnp.where(d == vmin, iota, NS), axis=1,
                      keepdims=True)
        masked = vmin > RADIUS * RADIUS
        here = liota == j
        idx_ref[0] = jnp.where(here, jnp.where(masked, -1, idx), idx_ref[0])
        dist_ref[0] = jnp.where(
            here, jnp.sqrt(jnp.maximum(vmin, 1e-9)) / RADIUS, dist_ref[0])
        return jnp.where(iota == idx, jnp.inf, d)

    def body(i2, _):
        d = d_scratch[...]
        d = extract(2 * i2, d)
        d_scratch[...] = extract(2 * i2 + 1, d)
        return 0

    jax.lax.fori_loop(0, K // 2, body, 0)


def _make_sc_gather(M, NT, n_workers, num_cores):
    per_w = M // n_workers
    CH = 2048
    n_chunks = per_w // CH
    steps = CH // 16
    f32 = jnp.float32
    mesh = plsc.VectorSubcoreMesh(core_axis_name="c", subcore_axis_name="s")

    @functools.partial(
        pl.kernel, mesh=mesh,
        compiler_params=pltpu.CompilerParams(needs_layout_passes=False),
        out_type=[jax.ShapeDtypeStruct((M,), f32)] * 3,
        scratch_types=(
            [pltpu.VMEM((NT,), f32) for _ in range(6)]
            + [pltpu.VMEM((CH,), jnp.int32)]
            + [pltpu.VMEM((CH,), f32) for _ in range(3)]
        ),
    )
    def sc_gather(gidx_hbm, xt_hbm, yt_hbm, zt_hbm, tnx_hbm, tny_hbm,
                  tnz_hbm, px_hbm, py_hbm, pz_hbm,
                  xt_v, yt_v, zt_v, tnx_v, tny_v, tnz_v,
                  idx_v, ox_v, oy_v, oz_v):
        wid = lax.axis_index("s") * num_cores + lax.axis_index("c")
        base = wid * per_w
        pltpu.sync_copy(xt_hbm, xt_v)
        pltpu.sync_copy(yt_hbm, yt_v)
        pltpu.sync_copy(zt_hbm, zt_v)
        pltpu.sync_copy(tnx_hbm, tnx_v)
        pltpu.sync_copy(tny_hbm, tny_v)
        pltpu.sync_copy(tnz_hbm, tnz_v)
        vio = lax.iota(jnp.int32, 16)

        def chunk_body(ci, _):
            cbase = base + ci * CH
            pltpu.sync_copy(gidx_hbm.at[pl.ds(cbase, CH)], idx_v)

            def step(j, _):
                off = j * 16
                iv = idx_v[pl.ds(off, 16)]
                tidx = ((cbase + off) + vio) >> 6
                ox_v[pl.ds(off, 16)] = (
                    plsc.load_gather(xt_v, [iv])
                    - plsc.load_gather(tnx_v, [tidx]))
                oy_v[pl.ds(off, 16)] = (
                    plsc.load_gather(yt_v, [iv])
                    - plsc.load_gather(tny_v, [tidx]))
                oz_v[pl.ds(off, 16)] = (
                    plsc.load_gather(zt_v, [iv])
                    - plsc.load_gather(tnz_v, [tidx]))
                return 0

            lax.fori_loop(0, steps, step, 0)
            pltpu.sync_copy(ox_v, px_hbm.at[pl.ds(cbase, CH)])
            pltpu.sync_copy(oy_v, py_hbm.at[pl.ds(cbase, CH)])
            pltpu.sync_copy(oz_v, pz_hbm.at[pl.ds(cbase, CH)])
            return 0

        lax.fori_loop(0, n_chunks, chunk_body, 0)

    return sc_gather


@jax.jit
def kernel(source_points, target_points):
    B, NT, _ = target_points.shape
    f32 = jnp.float32

    pad5 = jnp.zeros((B, NT, 5), f32)
    t8 = jnp.concatenate([target_points, pad5], axis=2)
    sT = jnp.transpose(source_points, (0, 2, 1))
    sT8 = jnp.concatenate([sT, jnp.zeros((B, 5, NS), f32)], axis=1)

    grid = (B, NT // TB)
    out_spec = pl.BlockSpec((1, TB, K), lambda b, tb: (b, tb, 0))
    idx, dist = pl.pallas_call(
        _topk_kernel,
        grid=grid,
        in_specs=[
            pl.BlockSpec((1, TB, 8), lambda b, tb: (b, tb, 0)),
            pl.BlockSpec((1, 8, NS), lambda b, tb: (b, 0, 0)),
        ],
        out_specs=[out_spec] * 2,
        out_shape=[
            jax.ShapeDtypeStruct((B, NT, K), jnp.int32),
            jax.ShapeDtypeStruct((B, NT, K), f32),
        ],
        scratch_shapes=[pltpu.VMEM((TB, NS), f32)],
    )(t8, sT8)

    bb = jnp.arange(B, dtype=jnp.int32).reshape(B, 1, 1)
    gidx = (bb * NS + (idx & (NS - 1))).reshape(-1)

    source_n = source_points / RADIUS
    target_n = target_points / RADIUS
    xt = source_n[..., 0].reshape(-1)
    yt = source_n[..., 1].reshape(-1)
    zt = source_n[..., 2].reshape(-1)
    tnx = target_n[..., 0].reshape(-1)
    tny = target_n[..., 1].reshape(-1)
    tnz = target_n[..., 2].reshape(-1)

    M = B * NT * K
    info = plsc.get_sparse_core_info()
    n_workers = info.num_cores * info.num_subcores
    sc_gather = _make_sc_gather(M, B * NS, n_workers, info.num_cores)
    px, py, pz = sc_gather(gidx, xt, yt, zt, tnx, tny, tnz)

    patches = jnp.stack(
        [px.reshape(B, NT, K), py.reshape(B, NT, K), pz.reshape(B, NT, K)],
        axis=-1)
    rad = jnp.full((B, 1, 1), RADIUS, f32)
    patches_size = jnp.full((B, NT), float(K), f32)
    return patches, idx, patches_size, rad, dist


# speedup vs baseline: 1.0721x; 1.0043x over previous
import functools

import jax
import jax.numpy as jnp
from jax import lax
from jax.experimental import pallas as pl
from jax.experimental.pallas import tpu as pltpu
from jax.experimental.pallas import tpu_sc as plsc

RADIUS = 100.0
K = 64
NS = 2048
TB = 256


def _topk_kernel(t8_ref, sT8_ref, idx_ref, dist_ref, d_scratch):
    t8 = t8_ref[0]
    sT8 = sT8_ref[0]

    mm = jnp.dot(t8.astype(jnp.bfloat16), sT8.astype(jnp.bfloat16),
                 preferred_element_type=jnp.float32)
    tx, ty, tz = t8[:, 0:1], t8[:, 1:2], t8[:, 2:3]
    r0 = (tx * tx + ty * ty) + tz * tz
    sx, sy, sz = sT8[0:1, :], sT8[1:2, :], sT8[2:3, :]
    r1 = (sx * sx + sy * sy) + sz * sz
    d_scratch[...] = (r0 - 2.0 * mm) + r1

    iota = jax.lax.broadcasted_iota(jnp.int32, (TB, NS), 1)
    liota = jax.lax.broadcasted_iota(jnp.int32, (TB, K), 1)

    def extract(j, d):
        vmin = jnp.min(d, axis=1, keepdims=True)
        idx = jnp.min(jnp.where(d == vmin, iota, NS), axis=1,
                      keepdims=True)
        masked = vmin > RADIUS * RADIUS
        here = liota == j
        idx_ref[0] = jnp.where(here, jnp.where(masked, -1, idx), idx_ref[0])
        dist_ref[0] = jnp.where(
            here, jnp.sqrt(jnp.maximum(vmin, 1e-9)) / RADIUS, dist_ref[0])
        return jnp.where(iota == idx, jnp.inf, d)

    def body(i4, _):
        d = d_scratch[...]
        d = extract(4 * i4, d)
        d = extract(4 * i4 + 1, d)
        d = extract(4 * i4 + 2, d)
        d_scratch[...] = extract(4 * i4 + 3, d)
        return 0

    jax.lax.fori_loop(0, K // 4, body, 0)


def _make_sc_gather(M, NT, n_workers, num_cores):
    per_w = M // n_workers
    CH = 2048
    n_chunks = per_w // CH
    steps = CH // 16
    f32 = jnp.float32
    mesh = plsc.VectorSubcoreMesh(core_axis_name="c", subcore_axis_name="s")

    @functools.partial(
        pl.kernel, mesh=mesh,
        compiler_params=pltpu.CompilerParams(needs_layout_passes=False),
        out_type=[jax.ShapeDtypeStruct((M,), f32)] * 3,
        scratch_types=(
            [pltpu.VMEM((NT,), f32) for _ in range(6)]
            + [pltpu.VMEM((CH,), jnp.int32)]
            + [pltpu.VMEM((CH,), f32) for _ in range(3)]
        ),
    )
    def sc_gather(gidx_hbm, xt_hbm, yt_hbm, zt_hbm, tnx_hbm, tny_hbm,
                  tnz_hbm, px_hbm, py_hbm, pz_hbm,
                  xt_v, yt_v, zt_v, tnx_v, tny_v, tnz_v,
                  idx_v, ox_v, oy_v, oz_v):
        wid = lax.axis_index("s") * num_cores + lax.axis_index("c")
        base = wid * per_w
        pltpu.sync_copy(xt_hbm, xt_v)
        pltpu.sync_copy(yt_hbm, yt_v)
        pltpu.sync_copy(zt_hbm, zt_v)
        pltpu.sync_copy(tnx_hbm, tnx_v)
        pltpu.sync_copy(tny_hbm, tny_v)
        pltpu.sync_copy(tnz_hbm, tnz_v)
        vio = lax.iota(jnp.int32, 16)

        def chunk_body(ci, _):
            cbase = base + ci * CH
            pltpu.sync_copy(gidx_hbm.at[pl.ds(cbase, CH)], idx_v)

            def step(j, _):
                off = j * 16
                iv = idx_v[pl.ds(off, 16)]
                tidx = ((cbase + off) + vio) >> 6
                ox_v[pl.ds(off, 16)] = (
                    plsc.load_gather(xt_v, [iv])
                    - plsc.load_gather(tnx_v, [tidx]))
                oy_v[pl.ds(off, 16)] = (
                    plsc.load_gather(yt_v, [iv])
                    - plsc.load_gather(tny_v, [tidx]))
                oz_v[pl.ds(off, 16)] = (
                    plsc.load_gather(zt_v, [iv])
                    - plsc.load_gather(tnz_v, [tidx]))
                return 0

            lax.fori_loop(0, steps, step, 0)
            pltpu.sync_copy(ox_v, px_hbm.at[pl.ds(cbase, CH)])
            pltpu.sync_copy(oy_v, py_hbm.at[pl.ds(cbase, CH)])
            pltpu.sync_copy(oz_v, pz_hbm.at[pl.ds(cbase, CH)])
            return 0

        lax.fori_loop(0, n_chunks, chunk_body, 0)

    return sc_gather


@jax.jit
def kernel(source_points, target_points):
    B, NT, _ = target_points.shape
    f32 = jnp.float32

    pad5 = jnp.zeros((B, NT, 5), f32)
    t8 = jnp.concatenate([target_points, pad5], axis=2)
    sT = jnp.transpose(source_points, (0, 2, 1))
    sT8 = jnp.concatenate([sT, jnp.zeros((B, 5, NS), f32)], axis=1)

    grid = (B, NT // TB)
    out_spec = pl.BlockSpec((1, TB, K), lambda b, tb: (b, tb, 0))
    idx, dist = pl.pallas_call(
        _topk_kernel,
        grid=grid,
        in_specs=[
            pl.BlockSpec((1, TB, 8), lambda b, tb: (b, tb, 0)),
            pl.BlockSpec((1, 8, NS), lambda b, tb: (b, 0, 0)),
        ],
        out_specs=[out_spec] * 2,
        out_shape=[
            jax.ShapeDtypeStruct((B, NT, K), jnp.int32),
            jax.ShapeDtypeStruct((B, NT, K), f32),
        ],
        scratch_shapes=[pltpu.VMEM((TB, NS), f32)],
    )(t8, sT8)

    bb = jnp.arange(B, dtype=jnp.int32).reshape(B, 1, 1)
    gidx = (bb * NS + (idx & (NS - 1))).reshape(-1)

    source_n = source_points / RADIUS
    target_n = target_points / RADIUS
    xt = source_n[..., 0].reshape(-1)
    yt = source_n[..., 1].reshape(-1)
    zt = source_n[..., 2].reshape(-1)
    tnx = target_n[..., 0].reshape(-1)
    tny = target_n[..., 1].reshape(-1)
    tnz = target_n[..., 2].reshape(-1)

    M = B * NT * K
    info = plsc.get_sparse_core_info()
    n_workers = info.num_cores * info.num_subcores
    sc_gather = _make_sc_gather(M, B * NS, n_workers, info.num_cores)
    px, py, pz = sc_gather(gidx, xt, yt, zt, tnx, tny, tnz)

    patches = jnp.stack(
        [px.reshape(B, NT, K), py.reshape(B, NT, K), pz.reshape(B, NT, K)],
        axis=-1)
    rad = jnp.full((B, 1, 1), RADIUS, f32)
    patches_size = jnp.full((B, NT), float(K), f32)
    return patches, idx, patches_size, rad, dist


# speedup vs baseline: 1.1871x; 1.1072x over previous
import functools

import jax
import jax.numpy as jnp
from jax import lax
from jax.experimental import pallas as pl
from jax.experimental.pallas import tpu as pltpu
from jax.experimental.pallas import tpu_sc as plsc

RADIUS = 100.0
K = 64
NS = 2048
TB = 512


def _topk_kernel(t8_ref, sT8_ref, idx_ref, dist_ref, d_scratch):
    t8 = t8_ref[0]
    sT8 = sT8_ref[0]

    mm = jnp.dot(t8.astype(jnp.bfloat16), sT8.astype(jnp.bfloat16),
                 preferred_element_type=jnp.float32)
    tx, ty, tz = t8[:, 0:1], t8[:, 1:2], t8[:, 2:3]
    r0 = (tx * tx + ty * ty) + tz * tz
    sx, sy, sz = sT8[0:1, :], sT8[1:2, :], sT8[2:3, :]
    r1 = (sx * sx + sy * sy) + sz * sz
    d_scratch[...] = (r0 - 2.0 * mm) + r1

    iota = jax.lax.broadcasted_iota(jnp.int32, (TB, NS), 1)
    liota = jax.lax.broadcasted_iota(jnp.int32, (TB, K), 1)

    def extract(j, d):
        vmin = jnp.min(d, axis=1, keepdims=True)
        idx = jnp.min(jnp.where(d == vmin, iota, NS), axis=1,
                      keepdims=True)
        masked = vmin > RADIUS * RADIUS
        here = liota == j
        idx_ref[0] = jnp.where(here, jnp.where(masked, -1, idx), idx_ref[0])
        dist_ref[0] = jnp.where(
            here, jnp.sqrt(jnp.maximum(vmin, 1e-9)) / RADIUS, dist_ref[0])
        return jnp.where(iota == idx, jnp.inf, d)

    def body(i4, _):
        d = d_scratch[...]
        d = extract(4 * i4, d)
        d = extract(4 * i4 + 1, d)
        d = extract(4 * i4 + 2, d)
        d_scratch[...] = extract(4 * i4 + 3, d)
        return 0

    jax.lax.fori_loop(0, K // 4, body, 0)


def _make_sc_gather(M, NT, n_workers, num_cores):
    per_w = M // n_workers
    CH = 2048
    n_chunks = per_w // CH
    steps = CH // 16
    f32 = jnp.float32
    mesh = plsc.VectorSubcoreMesh(core_axis_name="c", subcore_axis_name="s")

    @functools.partial(
        pl.kernel, mesh=mesh,
        compiler_params=pltpu.CompilerParams(needs_layout_passes=False),
        out_type=[jax.ShapeDtypeStruct((M,), f32)] * 3,
        scratch_types=(
            [pltpu.VMEM((NT,), f32) for _ in range(6)]
            + [pltpu.VMEM((CH,), jnp.int32)]
            + [pltpu.VMEM((CH,), f32) for _ in range(3)]
        ),
    )
    def sc_gather(gidx_hbm, xt_hbm, yt_hbm, zt_hbm, tnx_hbm, tny_hbm,
                  tnz_hbm, px_hbm, py_hbm, pz_hbm,
                  xt_v, yt_v, zt_v, tnx_v, tny_v, tnz_v,
                  idx_v, ox_v, oy_v, oz_v):
        wid = lax.axis_index("s") * num_cores + lax.axis_index("c")
        base = wid * per_w
        pltpu.sync_copy(xt_hbm, xt_v)
        pltpu.sync_copy(yt_hbm, yt_v)
        pltpu.sync_copy(zt_hbm, zt_v)
        pltpu.sync_copy(tnx_hbm, tnx_v)
        pltpu.sync_copy(tny_hbm, tny_v)
        pltpu.sync_copy(tnz_hbm, tnz_v)
        vio = lax.iota(jnp.int32, 16)

        def chunk_body(ci, _):
            cbase = base + ci * CH
            pltpu.sync_copy(gidx_hbm.at[pl.ds(cbase, CH)], idx_v)

            def step(j, _):
                off = j * 16
                iv = idx_v[pl.ds(off, 16)]
                tidx = ((cbase + off) + vio) >> 6
                ox_v[pl.ds(off, 16)] = (
                    plsc.load_gather(xt_v, [iv])
                    - plsc.load_gather(tnx_v, [tidx]))
                oy_v[pl.ds(off, 16)] = (
                    plsc.load_gather(yt_v, [iv])
                    - plsc.load_gather(tny_v, [tidx]))
                oz_v[pl.ds(off, 16)] = (
                    plsc.load_gather(zt_v, [iv])
                    - plsc.load_gather(tnz_v, [tidx]))
                return 0

            lax.fori_loop(0, steps, step, 0)
            pltpu.sync_copy(ox_v, px_hbm.at[pl.ds(cbase, CH)])
            pltpu.sync_copy(oy_v, py_hbm.at[pl.ds(cbase, CH)])
            pltpu.sync_copy(oz_v, pz_hbm.at[pl.ds(cbase, CH)])
            return 0

        lax.fori_loop(0, n_chunks, chunk_body, 0)

    return sc_gather


@jax.jit
def kernel(source_points, target_points):
    B, NT, _ = target_points.shape
    f32 = jnp.float32

    pad5 = jnp.zeros((B, NT, 5), f32)
    t8 = jnp.concatenate([target_points, pad5], axis=2)
    sT = jnp.transpose(source_points, (0, 2, 1))
    sT8 = jnp.concatenate([sT, jnp.zeros((B, 5, NS), f32)], axis=1)

    grid = (B, NT // TB)
    out_spec = pl.BlockSpec((1, TB, K), lambda b, tb: (b, tb, 0))
    idx, dist = pl.pallas_call(
        _topk_kernel,
        grid=grid,
        in_specs=[
            pl.BlockSpec((1, TB, 8), lambda b, tb: (b, tb, 0)),
            pl.BlockSpec((1, 8, NS), lambda b, tb: (b, 0, 0)),
        ],
        out_specs=[out_spec] * 2,
        out_shape=[
            jax.ShapeDtypeStruct((B, NT, K), jnp.int32),
            jax.ShapeDtypeStruct((B, NT, K), f32),
        ],
        scratch_shapes=[pltpu.VMEM((TB, NS), f32)],
    )(t8, sT8)

    bb = jnp.arange(B, dtype=jnp.int32).reshape(B, 1, 1)
    gidx = (bb * NS + (idx & (NS - 1))).reshape(-1)

    source_n = source_points / RADIUS
    target_n = target_points / RADIUS
    xt = source_n[..., 0].reshape(-1)
    yt = source_n[..., 1].reshape(-1)
    zt = source_n[..., 2].reshape(-1)
    tnx = target_n[..., 0].reshape(-1)
    tny = target_n[..., 1].reshape(-1)
    tnz = target_n[..., 2].reshape(-1)

    M = B * NT * K
    info = plsc.get_sparse_core_info()
    n_workers = info.num_cores * info.num_subcores
    sc_gather = _make_sc_gather(M, B * NS, n_workers, info.num_cores)
    px, py, pz = sc_gather(gidx, xt, yt, zt, tnx, tny, tnz)

    patches = jnp.stack(
        [px.reshape(B, NT, K), py.reshape(B, NT, K), pz.reshape(B, NT, K)],
        axis=-1)
    rad = jnp.full((B, 1, 1), RADIUS, f32)
    patches_size = jnp.full((B, NT), float(K), f32)
    return patches, idx, patches_size, rad, dist


# speedup vs baseline: 1.2019x; 1.0125x over previous
import functools

import jax
import jax.numpy as jnp
from jax import lax
from jax.experimental import pallas as pl
from jax.experimental.pallas import tpu as pltpu
from jax.experimental.pallas import tpu_sc as plsc

RADIUS = 100.0
K = 64
NS = 2048
TB = 1024


def _topk_kernel(t8_ref, sT8_ref, idx_ref, dist_ref, d_scratch):
    t8 = t8_ref[0]
    sT8 = sT8_ref[0]

    mm = jnp.dot(t8.astype(jnp.bfloat16), sT8.astype(jnp.bfloat16),
                 preferred_element_type=jnp.float32)
    tx, ty, tz = t8[:, 0:1], t8[:, 1:2], t8[:, 2:3]
    r0 = (tx * tx + ty * ty) + tz * tz
    sx, sy, sz = sT8[0:1, :], sT8[1:2, :], sT8[2:3, :]
    r1 = (sx * sx + sy * sy) + sz * sz
    d_scratch[...] = (r0 - 2.0 * mm) + r1

    iota = jax.lax.broadcasted_iota(jnp.int32, (TB, NS), 1)
    liota = jax.lax.broadcasted_iota(jnp.int32, (TB, K), 1)

    def extract(j, d):
        vmin = jnp.min(d, axis=1, keepdims=True)
        idx = jnp.min(jnp.where(d == vmin, iota, NS), axis=1,
                      keepdims=True)
        masked = vmin > RADIUS * RADIUS
        here = liota == j
        idx_ref[0] = jnp.where(here, jnp.where(masked, -1, idx), idx_ref[0])
        dist_ref[0] = jnp.where(
            here, jnp.sqrt(jnp.maximum(vmin, 1e-9)) / RADIUS, dist_ref[0])
        return jnp.where(iota == idx, jnp.inf, d)

    def body(i4, _):
        d = d_scratch[...]
        d = extract(4 * i4, d)
        d = extract(4 * i4 + 1, d)
        d = extract(4 * i4 + 2, d)
        d_scratch[...] = extract(4 * i4 + 3, d)
        return 0

    jax.lax.fori_loop(0, K // 4, body, 0)


def _make_sc_gather(M, NT, n_workers, num_cores):
    per_w = M // n_workers
    CH = 2048
    n_chunks = per_w // CH
    steps = CH // 16
    f32 = jnp.float32
    mesh = plsc.VectorSubcoreMesh(core_axis_name="c", subcore_axis_name="s")

    @functools.partial(
        pl.kernel, mesh=mesh,
        compiler_params=pltpu.CompilerParams(needs_layout_passes=False),
        out_type=[jax.ShapeDtypeStruct((M,), f32)] * 3,
        scratch_types=(
            [pltpu.VMEM((NT,), f32) for _ in range(6)]
            + [pltpu.VMEM((CH,), jnp.int32)]
            + [pltpu.VMEM((CH,), f32) for _ in range(3)]
        ),
    )
    def sc_gather(gidx_hbm, xt_hbm, yt_hbm, zt_hbm, tnx_hbm, tny_hbm,
                  tnz_hbm, px_hbm, py_hbm, pz_hbm,
                  xt_v, yt_v, zt_v, tnx_v, tny_v, tnz_v,
                  idx_v, ox_v, oy_v, oz_v):
        wid = lax.axis_index("s") * num_cores + lax.axis_index("c")
        base = wid * per_w
        pltpu.sync_copy(xt_hbm, xt_v)
        pltpu.sync_copy(yt_hbm, yt_v)
        pltpu.sync_copy(zt_hbm, zt_v)
        pltpu.sync_copy(tnx_hbm, tnx_v)
        pltpu.sync_copy(tny_hbm, tny_v)
        pltpu.sync_copy(tnz_hbm, tnz_v)
        vio = lax.iota(jnp.int32, 16)

        def chunk_body(ci, _):
            cbase = base + ci * CH
            pltpu.sync_copy(gidx_hbm.at[pl.ds(cbase, CH)], idx_v)

            def step(j, _):
                off = j * 16
                iv = idx_v[pl.ds(off, 16)]
                tidx = ((cbase + off) + vio) >> 6
                ox_v[pl.ds(off, 16)] = (
                    plsc.load_gather(xt_v, [iv])
                    - plsc.load_gather(tnx_v, [tidx]))
                oy_v[pl.ds(off, 16)] = (
                    plsc.load_gather(yt_v, [iv])
                    - plsc.load_gather(tny_v, [tidx]))
                oz_v[pl.ds(off, 16)] = (
                    plsc.load_gather(zt_v, [iv])
                    - plsc.load_gather(tnz_v, [tidx]))
                return 0

            lax.fori_loop(0, steps, step, 0)
            pltpu.sync_copy(ox_v, px_hbm.at[pl.ds(cbase, CH)])
            pltpu.sync_copy(oy_v, py_hbm.at[pl.ds(cbase, CH)])
            pltpu.sync_copy(oz_v, pz_hbm.at[pl.ds(cbase, CH)])
            return 0

        lax.fori_loop(0, n_chunks, chunk_body, 0)

    return sc_gather


@jax.jit
def kernel(source_points, target_points):
    B, NT, _ = target_points.shape
    f32 = jnp.float32

    pad5 = jnp.zeros((B, NT, 5), f32)
    t8 = jnp.concatenate([target_points, pad5], axis=2)
    sT = jnp.transpose(source_points, (0, 2, 1))
    sT8 = jnp.concatenate([sT, jnp.zeros((B, 5, NS), f32)], axis=1)

    grid = (B, NT // TB)
    out_spec = pl.BlockSpec((1, TB, K), lambda b, tb: (b, tb, 0))
    idx, dist = pl.pallas_call(
        _topk_kernel,
        grid=grid,
        in_specs=[
            pl.BlockSpec((1, TB, 8), lambda b, tb: (b, tb, 0)),
            pl.BlockSpec((1, 8, NS), lambda b, tb: (b, 0, 0)),
        ],
        out_specs=[out_spec] * 2,
        out_shape=[
            jax.ShapeDtypeStruct((B, NT, K), jnp.int32),
            jax.ShapeDtypeStruct((B, NT, K), f32),
        ],
        scratch_shapes=[pltpu.VMEM((TB, NS), f32)],
    )(t8, sT8)

    bb = jnp.arange(B, dtype=jnp.int32).reshape(B, 1, 1)
    gidx = (bb * NS + (idx & (NS - 1))).reshape(-1)

    source_n = source_points / RADIUS
    target_n = target_points / RADIUS
    xt = source_n[..., 0].reshape(-1)
    yt = source_n[..., 1].reshape(-1)
    zt = source_n[..., 2].reshape(-1)
    tnx = target_n[..., 0].reshape(-1)
    tny = target_n[..., 1].reshape(-1)
    tnz = target_n[..., 2].reshape(-1)

    M = B * NT * K
    info = plsc.get_sparse_core_info()
    n_workers = info.num_cores * info.num_subcores
    sc_gather = _make_sc_gather(M, B * NS, n_workers, info.num_cores)
    px, py, pz = sc_gather(gidx, xt, yt, zt, tnx, tny, tnz)

    patches = jnp.stack(
        [px.reshape(B, NT, K), py.reshape(B, NT, K), pz.reshape(B, NT, K)],
        axis=-1)
    rad = jnp.full((B, 1, 1), RADIUS, f32)
    patches_size = jnp.full((B, NT), float(K), f32)
    return patches, idx, patches_size, rad, dist


# speedup vs baseline: 1.2059x; 1.0033x over previous
import functools

import jax
import jax.numpy as jnp
from jax import lax
from jax.experimental import pallas as pl
from jax.experimental.pallas import tpu as pltpu
from jax.experimental.pallas import tpu_sc as plsc

RADIUS = 100.0
K = 64
NS = 2048
TB = 2048


def _topk_kernel(t8_ref, sT8_ref, idx_ref, dist_ref, d_scratch):
    t8 = t8_ref[0]
    sT8 = sT8_ref[0]

    mm = jnp.dot(t8.astype(jnp.bfloat16), sT8.astype(jnp.bfloat16),
                 preferred_element_type=jnp.float32)
    tx, ty, tz = t8[:, 0:1], t8[:, 1:2], t8[:, 2:3]
    r0 = (tx * tx + ty * ty) + tz * tz
    sx, sy, sz = sT8[0:1, :], sT8[1:2, :], sT8[2:3, :]
    r1 = (sx * sx + sy * sy) + sz * sz
    d_scratch[...] = (r0 - 2.0 * mm) + r1

    iota = jax.lax.broadcasted_iota(jnp.int32, (TB, NS), 1)
    liota = jax.lax.broadcasted_iota(jnp.int32, (TB, K), 1)

    def extract(j, d):
        vmin = jnp.min(d, axis=1, keepdims=True)
        idx = jnp.min(jnp.where(d == vmin, iota, NS), axis=1,
                      keepdims=True)
        masked = vmin > RADIUS * RADIUS
        here = liota == j
        idx_ref[0] = jnp.where(here, jnp.where(masked, -1, idx), idx_ref[0])
        dist_ref[0] = jnp.where(
            here, jnp.sqrt(jnp.maximum(vmin, 1e-9)) / RADIUS, dist_ref[0])
        return jnp.where(iota == idx, jnp.inf, d)

    def body(i4, _):
        d = d_scratch[...]
        d = extract(4 * i4, d)
        d = extract(4 * i4 + 1, d)
        d = extract(4 * i4 + 2, d)
        d_scratch[...] = extract(4 * i4 + 3, d)
        return 0

    jax.lax.fori_loop(0, K // 4, body, 0)


def _make_sc_gather(M, NT, n_workers, num_cores):
    per_w = M // n_workers
    CH = 2048
    n_chunks = per_w // CH
    steps = CH // 16
    f32 = jnp.float32
    mesh = plsc.VectorSubcoreMesh(core_axis_name="c", subcore_axis_name="s")

    @functools.partial(
        pl.kernel, mesh=mesh,
        compiler_params=pltpu.CompilerParams(needs_layout_passes=False),
        out_type=[jax.ShapeDtypeStruct((M,), f32)] * 3,
        scratch_types=(
            [pltpu.VMEM((NT,), f32) for _ in range(6)]
            + [pltpu.VMEM((CH,), jnp.int32)]
            + [pltpu.VMEM((CH,), f32) for _ in range(3)]
        ),
    )
    def sc_gather(gidx_hbm, xt_hbm, yt_hbm, zt_hbm, tnx_hbm, tny_hbm,
                  tnz_hbm, px_hbm, py_hbm, pz_hbm,
                  xt_v, yt_v, zt_v, tnx_v, tny_v, tnz_v,
                  idx_v, ox_v, oy_v, oz_v):
        wid = lax.axis_index("s") * num_cores + lax.axis_index("c")
        base = wid * per_w
        pltpu.sync_copy(xt_hbm, xt_v)
        pltpu.sync_copy(yt_hbm, yt_v)
        pltpu.sync_copy(zt_hbm, zt_v)
        pltpu.sync_copy(tnx_hbm, tnx_v)
        pltpu.sync_copy(tny_hbm, tny_v)
        pltpu.sync_copy(tnz_hbm, tnz_v)
        vio = lax.iota(jnp.int32, 16)

        def chunk_body(ci, _):
            cbase = base + ci * CH
            pltpu.sync_copy(gidx_hbm.at[pl.ds(cbase, CH)], idx_v)

            def step(j, _):
                off = j * 16
                iv = idx_v[pl.ds(off, 16)]
                tidx = ((cbase + off) + vio) >> 6
                ox_v[pl.ds(off, 16)] = (
                    plsc.load_gather(xt_v, [iv])
                    - plsc.load_gather(tnx_v, [tidx]))
                oy_v[pl.ds(off, 16)] = (
                    plsc.load_gather(yt_v, [iv])
                    - plsc.load_gather(tny_v, [tidx]))
                oz_v[pl.ds(off, 16)] = (
                    plsc.load_gather(zt_v, [iv])
                    - plsc.load_gather(tnz_v, [tidx]))
                return 0

            lax.fori_loop(0, steps, step, 0)
            pltpu.sync_copy(ox_v, px_hbm.at[pl.ds(cbase, CH)])
            pltpu.sync_copy(oy_v, py_hbm.at[pl.ds(cbase, CH)])
            pltpu.sync_copy(oz_v, pz_hbm.at[pl.ds(cbase, CH)])
            return 0

        lax.fori_loop(0, n_chunks, chunk_body, 0)

    return sc_gather


@jax.jit
def kernel(source_points, target_points):
    B, NT, _ = target_points.shape
    f32 = jnp.float32

    pad5 = jnp.zeros((B, NT, 5), f32)
    t8 = jnp.concatenate([target_points, pad5], axis=2)
    sT = jnp.transpose(source_points, (0, 2, 1))
    sT8 = jnp.concatenate([sT, jnp.zeros((B, 5, NS), f32)], axis=1)

    grid = (B, NT // TB)
    out_spec = pl.BlockSpec((1, TB, K), lambda b, tb: (b, tb, 0))
    idx, dist = pl.pallas_call(
        _topk_kernel,
        grid=grid,
        in_specs=[
            pl.BlockSpec((1, TB, 8), lambda b, tb: (b, tb, 0)),
            pl.BlockSpec((1, 8, NS), lambda b, tb: (b, 0, 0)),
        ],
        out_specs=[out_spec] * 2,
        out_shape=[
            jax.ShapeDtypeStruct((B, NT, K), jnp.int32),
            jax.ShapeDtypeStruct((B, NT, K), f32),
        ],
        scratch_shapes=[pltpu.VMEM((TB, NS), f32)],
    )(t8, sT8)

    bb = jnp.arange(B, dtype=jnp.int32).reshape(B, 1, 1)
    gidx = (bb * NS + (idx & (NS - 1))).reshape(-1)

    source_n = source_points / RADIUS
    target_n = target_points / RADIUS
    xt = source_n[..., 0].reshape(-1)
    yt = source_n[..., 1].reshape(-1)
    zt = source_n[..., 2].reshape(-1)
    tnx = target_n[..., 0].reshape(-1)
    tny = target_n[..., 1].reshape(-1)
    tnz = target_n[..., 2].reshape(-1)

    M = B * NT * K
    info = plsc.get_sparse_core_info()
    n_workers = info.num_cores * info.num_subcores
    sc_gather = _make_sc_gather(M, B * NS, n_workers, info.num_cores)
    px, py, pz = sc_gather(gidx, xt, yt, zt, tnx, tny, tnz)

    patches = jnp.stack(
        [px.reshape(B, NT, K), py.reshape(B, NT, K), pz.reshape(B, NT, K)],
        axis=-1)
    rad = jnp.full((B, 1, 1), RADIUS, f32)
    patches_size = jnp.full((B, NT), float(K), f32)
    return patches, idx, patches_size, rad, dist
